# baseline (speedup 1.0000x reference)
import functools

import jax
import jax.numpy as jnp
from jax import lax
from jax.experimental import pallas as pl
from jax.experimental.pallas import tpu as pltpu
from jax.experimental.pallas import tpu_sc as plsc

B = 16384
L = 200
DIM = 64
TOTAL = B * L
NW = 32
ROWS_PER_W = TOTAL // NW
CHUNK = 1024
SUB = 128
N_SUB = CHUNK // SUB
N_CHUNKS = ROWS_PER_W // CHUNK


def _make_kernel():
    mesh = plsc.VectorSubcoreMesh(core_axis_name="c", subcore_axis_name="s")

    @functools.partial(
        pl.kernel,
        out_type=jax.ShapeDtypeStruct((TOTAL, DIM), jnp.float32),
        mesh=mesh,
        scratch_types=[
            pltpu.VMEM((CHUNK,), jnp.int32),
            pltpu.VMEM((CHUNK, DIM), jnp.float32),
            pltpu.SemaphoreType.DMA,
        ],
        compiler_params=pltpu.CompilerParams(use_tc_tiling_on_sc=False),
    )
    def gather_kernel(idx_hbm, table_hbm, out_hbm, idx_v, rows_v, sem):
        wid = lax.axis_index("s") * 2 + lax.axis_index("c")
        wbase = wid * ROWS_PER_W

        def chunk_body(g, carry):
            base = wbase + g * CHUNK
            pltpu.sync_copy(idx_hbm.at[pl.ds(base, CHUNK)], idx_v)
            copies = []
            for j in range(N_SUB):
                cp = pltpu.async_copy(
                    table_hbm.at[idx_v.at[pl.ds(j * SUB, SUB)]],
                    rows_v.at[pl.ds(j * SUB, SUB), :],
                    sem,
                )
                copies.append(cp)
            for cp in copies:
                cp.wait()
            pltpu.sync_copy(rows_v, out_hbm.at[pl.ds(base, CHUNK)])
            return carry

        lax.fori_loop(0, N_CHUNKS, chunk_body, 0)

    return gather_kernel


_gather = _make_kernel()


@jax.jit
def kernel(x, weight):
    idx = x.reshape(TOTAL).astype(jnp.int32)
    out = _gather(idx, weight)
    return out.reshape(B, L, DIM)


# speedup vs baseline: 1.0117x; 1.0117x over previous
import functools

import jax
import jax.numpy as jnp
from jax import lax
from jax.experimental import pallas as pl
from jax.experimental.pallas import tpu as pltpu
from jax.experimental.pallas import tpu_sc as plsc

B = 16384
L = 200
DIM = 64
TOTAL = B * L
NW = 32
ROWS_PER_W = TOTAL // NW
CHUNK = 640
SUB = 128
N_SUB = CHUNK // SUB
N_CHUNKS = ROWS_PER_W // CHUNK
N_PAIRS = N_CHUNKS // 2


def _make_kernel():
    mesh = plsc.VectorSubcoreMesh(core_axis_name="c", subcore_axis_name="s")

    @functools.partial(
        pl.kernel,
        out_type=jax.ShapeDtypeStruct((TOTAL, DIM), jnp.float32),
        mesh=mesh,
        scratch_types=[
            pltpu.VMEM((CHUNK,), jnp.int32),
            pltpu.VMEM((CHUNK,), jnp.int32),
            pltpu.VMEM((CHUNK, DIM), jnp.float32),
            pltpu.VMEM((CHUNK, DIM), jnp.float32),
            pltpu.SemaphoreType.DMA,
            pltpu.SemaphoreType.DMA,
            pltpu.SemaphoreType.DMA,
            pltpu.SemaphoreType.DMA,
        ],
        compiler_params=pltpu.CompilerParams(use_tc_tiling_on_sc=False),
    )
    def gather_kernel(idx_hbm, table_hbm, out_hbm, idx0, idx1, rows0, rows1,
                      gsem0, gsem1, ssem0, ssem1):
        wid = lax.axis_index("s") * 2 + lax.axis_index("c")
        wbase = wid * ROWS_PER_W

        idx_v = (idx0, idx1)
        rows_v = (rows0, rows1)
        gsem = (gsem0, gsem1)
        ssem = (ssem0, ssem1)

        def start_gathers(g, b):
            base = wbase + g * CHUNK
            pltpu.sync_copy(idx_hbm.at[pl.ds(base, CHUNK)], idx_v[b])
            for j in range(N_SUB):
                pltpu.async_copy(
                    table_hbm.at[idx_v[b].at[pl.ds(j * SUB, SUB)]],
                    rows_v[b].at[pl.ds(j * SUB, SUB), :],
                    gsem[b],
                )

        def wait_chunk(sem, b):
            pltpu.make_async_copy(
                out_hbm.at[pl.ds(0, CHUNK)], rows_v[b], sem
            ).wait()

        def start_store(g, b):
            base = wbase + g * CHUNK
            pltpu.async_copy(rows_v[b], out_hbm.at[pl.ds(base, CHUNK)], ssem[b])

        def step(g, b, nb, first, last):
            if not last:
                if not first:
                    wait_chunk(ssem[nb], nb)
                start_gathers(g + 1, nb)
            wait_chunk(gsem[b], b)
            start_store(g, b)

        def pair_body(p, carry):
            g = 2 * p

            @pl.when(p > 0)
            def _():
                step(g, 0, 1, False, False)

            @pl.when(p == 0)
            def _():
                step(g, 0, 1, True, False)

            @pl.when(p < N_PAIRS - 1)
            def _():
                step(g + 1, 1, 0, False, False)

            @pl.when(p == N_PAIRS - 1)
            def _():
                step(g + 1, 1, 0, False, True)

            return carry

        start_gathers(0, 0)
        lax.fori_loop(0, N_PAIRS, pair_body, 0)
        wait_chunk(ssem[0], 0)
        wait_chunk(ssem[1], 1)

    return gather_kernel


_gather = _make_kernel()


@jax.jit
def kernel(x, weight):
    idx = x.reshape(TOTAL).astype(jnp.int32)
    out = _gather(idx, weight)
    return out.reshape(B, L, DIM)


# speedup vs baseline: 1.0145x; 1.0028x over previous
import functools

import jax
import jax.numpy as jnp
from jax import lax
from jax.experimental import pallas as pl
from jax.experimental.pallas import tpu as pltpu
from jax.experimental.pallas import tpu_sc as plsc

B = 16384
L = 200
DIM = 64
TOTAL = B * L
NW = 32
ROWS_PER_W = TOTAL // NW
CHUNK = 640
SUB = 128
N_SUB = CHUNK // SUB
N_CHUNKS = ROWS_PER_W // CHUNK
N_PAIRS = N_CHUNKS // 2


def _make_kernel():
    mesh = plsc.VectorSubcoreMesh(core_axis_name="c", subcore_axis_name="s")

    @functools.partial(
        pl.kernel,
        out_type=jax.ShapeDtypeStruct((TOTAL, DIM), jnp.float32),
        mesh=mesh,
        scratch_types=[
            pltpu.VMEM((CHUNK,), jnp.int32),
            pltpu.VMEM((CHUNK,), jnp.int32),
            pltpu.VMEM((CHUNK, DIM), jnp.float32),
            pltpu.VMEM((CHUNK, DIM), jnp.float32),
            pltpu.SemaphoreType.DMA,
            pltpu.SemaphoreType.DMA,
            pltpu.SemaphoreType.DMA,
            pltpu.SemaphoreType.DMA,
        ],
        compiler_params=pltpu.CompilerParams(
            use_tc_tiling_on_sc=False,
            skip_device_barrier=True,
        ),
    )
    def gather_kernel(idx_hbm, table_hbm, out_hbm, idx0, idx1, rows0, rows1,
                      gsem0, gsem1, ssem0, ssem1):
        wid = lax.axis_index("s") * 2 + lax.axis_index("c")
        wbase = wid * ROWS_PER_W

        idx_v = (idx0, idx1)
        rows_v = (rows0, rows1)
        gsem = (gsem0, gsem1)
        ssem = (ssem0, ssem1)

        def start_gathers(g, b):
            base = wbase + g * CHUNK
            pltpu.sync_copy(idx_hbm.at[pl.ds(base, CHUNK)], idx_v[b])
            for j in range(N_SUB):
                pltpu.async_copy(
                    table_hbm.at[idx_v[b].at[pl.ds(j * SUB, SUB)]],
                    rows_v[b].at[pl.ds(j * SUB, SUB), :],
                    gsem[b],
                )

        def wait_chunk(sem, b):
            pltpu.make_async_copy(
                out_hbm.at[pl.ds(0, CHUNK)], rows_v[b], sem
            ).wait()

        def start_store(g, b):
            base = wbase + g * CHUNK
            pltpu.async_copy(rows_v[b], out_hbm.at[pl.ds(base, CHUNK)], ssem[b])

        def step(g, b, nb, first, last):
            if not last:
                if not first:
                    wait_chunk(ssem[nb], nb)
                start_gathers(g + 1, nb)
            wait_chunk(gsem[b], b)
            start_store(g, b)

        def pair_body(p, carry):
            g = 2 * p

            @pl.when(p > 0)
            def _():
                step(g, 0, 1, False, False)

            @pl.when(p == 0)
            def _():
                step(g, 0, 1, True, False)

            @pl.when(p < N_PAIRS - 1)
            def _():
                step(g + 1, 1, 0, False, False)

            @pl.when(p == N_PAIRS - 1)
            def _():
                step(g + 1, 1, 0, False, True)

            return carry

        start_gathers(0, 0)
        lax.fori_loop(0, N_PAIRS, pair_body, 0)
        wait_chunk(ssem[0], 0)
        wait_chunk(ssem[1], 1)

    return gather_kernel


_gather = _make_kernel()


@jax.jit
def kernel(x, weight):
    idx = x.reshape(TOTAL).astype(jnp.int32)
    out = _gather(idx, weight)
    return out.reshape(B, L, DIM)


# speedup vs baseline: 1.6646x; 1.6408x over previous
import functools

import jax
import jax.numpy as jnp
from jax import lax
from jax.experimental import pallas as pl
from jax.experimental.pallas import tpu as pltpu
from jax.experimental.pallas import tpu_sc as plsc

B = 16384
L = 200
DIM = 64
PDIM = 128
TOTAL = B * L
NW = 32
ROWS_PER_W = TOTAL // NW
CHUNK = 640
SUB = 128
N_SUB = CHUNK // SUB
N_CHUNKS = ROWS_PER_W // CHUNK
N_PAIRS = N_CHUNKS // 2


def _make_kernel():
    mesh = plsc.VectorSubcoreMesh(core_axis_name="c", subcore_axis_name="s")

    @functools.partial(
        pl.kernel,
        out_type=jax.ShapeDtypeStruct((TOTAL, PDIM), jnp.float32),
        mesh=mesh,
        scratch_types=[
            pltpu.VMEM((CHUNK,), jnp.int32),
            pltpu.VMEM((CHUNK,), jnp.int32),
            pltpu.VMEM((CHUNK, DIM), jnp.float32),
            pltpu.VMEM((CHUNK, DIM), jnp.float32),
            pltpu.SemaphoreType.DMA,
            pltpu.SemaphoreType.DMA,
            pltpu.SemaphoreType.DMA,
            pltpu.SemaphoreType.DMA,
        ],
        compiler_params=pltpu.CompilerParams(
            use_tc_tiling_on_sc=False,
            skip_device_barrier=True,
        ),
    )
    def gather_kernel(idx_hbm, table_hbm, out_hbm, idx0, idx1, rows0, rows1,
                      gsem0, gsem1, ssem0, ssem1):
        wid = lax.axis_index("s") * 2 + lax.axis_index("c")
        wbase = wid * ROWS_PER_W

        idx_v = (idx0, idx1)
        rows_v = (rows0, rows1)
        gsem = (gsem0, gsem1)
        ssem = (ssem0, ssem1)

        def start_gathers(g, b):
            base = wbase + g * CHUNK
            pltpu.sync_copy(idx_hbm.at[pl.ds(base, CHUNK)], idx_v[b])
            for j in range(N_SUB):
                pltpu.async_copy(
                    table_hbm.at[idx_v[b].at[pl.ds(j * SUB, SUB)]],
                    rows_v[b].at[pl.ds(j * SUB, SUB), :],
                    gsem[b],
                )

        def wait_chunk(sem, b):
            pltpu.make_async_copy(
                out_hbm.at[pl.ds(0, CHUNK), pl.ds(0, DIM)], rows_v[b], sem
            ).wait()

        def start_store(g, b):
            base = wbase + g * CHUNK
            pltpu.async_copy(
                rows_v[b],
                out_hbm.at[pl.ds(base, CHUNK), pl.ds(0, DIM)],
                ssem[b],
            )

        def step(g, b, nb, first, last):
            if not last:
                if not first:
                    wait_chunk(ssem[nb], nb)
                start_gathers(g + 1, nb)
            wait_chunk(gsem[b], b)
            start_store(g, b)

        def pair_body(p, carry):
            g = 2 * p

            @pl.when(p > 0)
            def _():
                step(g, 0, 1, False, False)

            @pl.when(p == 0)
            def _():
                step(g, 0, 1, True, False)

            @pl.when(p < N_PAIRS - 1)
            def _():
                step(g + 1, 1, 0, False, False)

            @pl.when(p == N_PAIRS - 1)
            def _():
                step(g + 1, 1, 0, False, True)

            return carry

        start_gathers(0, 0)
        lax.fori_loop(0, N_PAIRS, pair_body, 0)
        wait_chunk(ssem[0], 0)
        wait_chunk(ssem[1], 1)

    return gather_kernel


_gather = _make_kernel()


@jax.jit
def kernel(x, weight):
    idx = x.reshape(TOTAL).astype(jnp.int32)
    out = _gather(idx, weight)
    return out[:, :DIM].reshape(B, L, DIM)
